# baseline (speedup 1.0000x reference)
import functools

import jax
import jax.numpy as jnp
from jax import lax
from jax.experimental import pallas as pl
from jax.experimental.pallas import tpu as pltpu
from jax.experimental.pallas import tpu_sc as plsc

_LANES = 16


def _make_sc_lookup(batch, num_ids):
    info = plsc.get_sparse_core_info()
    nc, ns = 1, 8
    nw = nc * ns
    b_per_w = batch // nw
    mesh = plsc.VectorSubcoreMesh(
        core_axis_name="c", subcore_axis_name="s", num_cores=1, num_subcores=8
    )

    n_chunks = 2
    chunk = b_per_w // n_chunks

    @functools.partial(
        pl.kernel,
        mesh=mesh,
        out_type=jax.ShapeDtypeStruct((batch,), jnp.float32),
        scratch_types=[
            pltpu.VMEM((b_per_w,), jnp.int32),
            pltpu.VMEM((b_per_w,), jnp.float32),
            [pltpu.SemaphoreType.DMA] * n_chunks,
            [pltpu.SemaphoreType.DMA] * n_chunks,
            [pltpu.SemaphoreType.DMA] * n_chunks,
        ],
    )
    def lookup(ids_hbm, table_hbm, out_hbm, idx_v, rows_v, isems, gsems, osems):
        wid = lax.axis_index("s") * nc + lax.axis_index("c")
        base = wid * b_per_w
        idx_copies = [
            pltpu.async_copy(
                ids_hbm.at[pl.ds(base + j * chunk, chunk)],
                idx_v.at[pl.ds(j * chunk, chunk)],
                isems[j],
            )
            for j in range(n_chunks)
        ]
        gathers = []
        for j in range(n_chunks):
            idx_copies[j].wait()
            gathers.append(
                pltpu.async_copy(
                    table_hbm.at[idx_v.at[pl.ds(j * chunk, chunk)]],
                    rows_v.at[pl.ds(j * chunk, chunk)],
                    gsems[j],
                )
            )
        outs = []
        for j in range(n_chunks):
            gathers[j].wait()

            def _zero(i, _, j=j):
                sl = pl.ds(j * chunk + i * _LANES, _LANES)
                w = rows_v[sl]
                rows_v[sl] = w - w
                return 0

            lax.fori_loop(0, chunk // _LANES, _zero, 0)
            outs.append(
                pltpu.async_copy(
                    rows_v.at[pl.ds(j * chunk, chunk)],
                    out_hbm.at[pl.ds(base + j * chunk, chunk)],
                    osems[j],
                )
            )
        for o in outs:
            o.wait()

    return lookup


def kernel(witness_ids, witness_weight):
    batch = witness_ids.shape[0]
    num_ids = witness_weight.shape[0]
    ids = witness_ids.astype(jnp.int32)
    table = witness_weight.reshape(num_ids)
    out = _make_sc_lookup(batch, num_ids)(ids, table)
    return out.reshape(batch, 1)


# speedup vs baseline: 1.0234x; 1.0234x over previous
import functools

import jax
import jax.numpy as jnp
from jax import lax
from jax.experimental import pallas as pl
from jax.experimental.pallas import tpu as pltpu
from jax.experimental.pallas import tpu_sc as plsc

_LANES = 16
_N_CHUNKS = 2


def _make_sc_lookup(batch, num_ids):
    info = plsc.get_sparse_core_info()
    ns = info.num_subcores
    b_per_w = batch // ns
    chunk = b_per_w // _N_CHUNKS
    assert chunk % (8 * _LANES) == 0
    mesh = plsc.VectorSubcoreMesh(
        core_axis_name="c", subcore_axis_name="s", num_cores=1
    )

    @functools.partial(
        pl.kernel,
        mesh=mesh,
        out_type=jax.ShapeDtypeStruct((batch,), jnp.float32),
        scratch_types=[
            pltpu.VMEM((b_per_w,), jnp.int32),
            pltpu.VMEM((b_per_w,), jnp.float32),
            [pltpu.SemaphoreType.DMA] * _N_CHUNKS,
            [pltpu.SemaphoreType.DMA] * _N_CHUNKS,
            [pltpu.SemaphoreType.DMA] * _N_CHUNKS,
        ],
    )
    def lookup(ids_hbm, table_hbm, out_hbm, idx_v, rows_v, isems, gsems, osems):
        base = lax.axis_index("s") * b_per_w
        idx_copies = [
            pltpu.async_copy(
                ids_hbm.at[pl.ds(base + j * chunk, chunk)],
                idx_v.at[pl.ds(j * chunk, chunk)],
                isems[j],
            )
            for j in range(_N_CHUNKS)
        ]
        gathers = []
        for j in range(_N_CHUNKS):
            idx_copies[j].wait()
            gathers.append(
                pltpu.async_copy(
                    table_hbm.at[idx_v.at[pl.ds(j * chunk, chunk)]],
                    rows_v.at[pl.ds(j * chunk, chunk)],
                    gsems[j],
                )
            )
        outs = []
        for j in range(_N_CHUNKS):
            gathers[j].wait()
            for i in range(chunk // _LANES):
                sl = pl.ds(j * chunk + i * _LANES, _LANES)
                w = rows_v[sl]
                rows_v[sl] = w - w
            outs.append(
                pltpu.async_copy(
                    rows_v.at[pl.ds(j * chunk, chunk)],
                    out_hbm.at[pl.ds(base + j * chunk, chunk)],
                    osems[j],
                )
            )
        for o in outs:
            o.wait()

    return lookup


def kernel(witness_ids, witness_weight):
    batch = witness_ids.shape[0]
    num_ids = witness_weight.shape[0]
    ids = witness_ids.astype(jnp.int32)
    table = witness_weight.reshape(num_ids)
    out = _make_sc_lookup(batch, num_ids)(ids, table)
    return out.reshape(batch, 1)
